# baseline (speedup 1.0000x reference)
import functools

import jax
import jax.numpy as jnp
from jax import lax
from jax.experimental import pallas as pl
from jax.experimental.pallas import tpu as pltpu
from jax.experimental.pallas import tpu_sc as plsc

B = 16384
D = 64

_info = plsc.get_sparse_core_info()
_NC = _info.num_cores
_NS = _info.num_subcores
_L = _info.num_lanes
_NW = _NC * _NS
_BPW = B // _NW
_NSLOT = 8
_NG = _BPW // _L

_mesh = plsc.VectorSubcoreMesh(core_axis_name="c", subcore_axis_name="s")

_SHUF_DNUMS = lax.GatherDimensionNumbers(
    offset_dims=(), collapsed_slice_dims=(0,), start_index_map=(0,))


def _lane_shuffle(x, idx):
    return lax.gather(x, idx[:, None], _SHUF_DNUMS, slice_sizes=(1,),
                      mode=lax.GatherScatterMode.PROMISE_IN_BOUNDS)


@functools.partial(
    pl.kernel,
    mesh=_mesh,
    compiler_params=pltpu.CompilerParams(
        needs_layout_passes=False, skip_device_barrier=True),
    out_type=jax.ShapeDtypeStruct((B,), jnp.float32),
    scratch_types=[
        pltpu.VMEM((_BPW,), jnp.int32),
        pltpu.VMEM((_BPW,), jnp.int32),
        pltpu.VMEM((_NSLOT, 1, D), jnp.float32),
        pltpu.VMEM((_NSLOT, 1, D), jnp.float32),
        pltpu.VMEM((_BPW,), jnp.float32),
        [pltpu.SemaphoreType.DMA] * _NSLOT,
        [pltpu.SemaphoreType.DMA] * _NSLOT,
    ],
)
def _mf_kernel(u_hbm, i_hbm, U_hbm, V_hbm, out_hbm,
               uidx, vidx, ublk, vblk, outv, usem, vsem):
    wid = lax.axis_index("s") * _NC + lax.axis_index("c")
    base = wid * _BPW

    pltpu.sync_copy(u_hbm.at[pl.ds(base, _BPW)], uidx)
    pltpu.sync_copy(i_hbm.at[pl.ds(base, _BPW)], vidx)

    lanes = lax.iota(jnp.int32, _L)

    def _extract(vec, t):
        return jnp.sum(jnp.where(lanes == t, vec, 0))

    def issue(g, t, slot):
        ru = _extract(uidx[pl.ds(g * _L, _L)], t)
        rv = _extract(vidx[pl.ds(g * _L, _L)], t)
        pltpu.async_copy(U_hbm.at[ru >> 3, pl.ds(ru & 7, 1)],
                         ublk.at[slot], usem[slot])
        pltpu.async_copy(V_hbm.at[pl.ds(rv, 1)],
                         vblk.at[slot], vsem[slot])

    for t in range(_NSLOT):
        issue(0, t, t)

    def body_g(g, carry):
        acc = jnp.zeros((_L,), jnp.float32)
        for t in range(_L):
            slot = t % _NSLOT
            pltpu.make_async_copy(
                U_hbm.at[0, pl.ds(0, 1)], ublk.at[slot], usem[slot]).wait()
            pltpu.make_async_copy(
                V_hbm.at[pl.ds(0, 1)], vblk.at[slot], vsem[slot]).wait()
            p = ublk[slot, 0, pl.ds(0, _L)] * vblk[slot, 0, pl.ds(0, _L)]
            for c in range(1, D // _L):
                p += (ublk[slot, 0, pl.ds(c * _L, _L)]
                      * vblk[slot, 0, pl.ds(c * _L, _L)])
            for h in (8, 4, 2, 1):
                p = p + _lane_shuffle(p, lanes ^ h)
            acc = jnp.where(lanes == t, p, acc)

            if t < _NSLOT:
                issue(g, t + _NSLOT, slot)
            else:
                @pl.when(g < _NG - 1)
                def _():
                    issue(g + 1, t - _NSLOT, slot)

        outv[pl.ds(g * _L, _L)] = acc
        return carry

    lax.fori_loop(0, _NG, body_g, 0)
    pltpu.sync_copy(outv, out_hbm.at[pl.ds(base, _BPW)])


def kernel(u, i, U_emb, V_emb):
    U3 = U_emb.reshape(125000, 8, D)
    return _mf_kernel(u.astype(jnp.int32), i.astype(jnp.int32), U3, V_emb)


# speedup vs baseline: 1.0764x; 1.0764x over previous
import functools

import jax
import jax.numpy as jnp
from jax import lax
from jax.experimental import pallas as pl
from jax.experimental.pallas import tpu as pltpu
from jax.experimental.pallas import tpu_sc as plsc

B = 16384
D = 64

_info = plsc.get_sparse_core_info()
_NC = _info.num_cores
_NS = _info.num_subcores
_L = _info.num_lanes
_NW = _NC * _NS
_BPW = B // _NW
_NSLOT = 8
_NG = _BPW // _L

_mesh = plsc.VectorSubcoreMesh(core_axis_name="c", subcore_axis_name="s")

_SHUF_DNUMS = lax.GatherDimensionNumbers(
    offset_dims=(), collapsed_slice_dims=(0,), start_index_map=(0,))


def _lane_shuffle(x, idx):
    return lax.gather(x, idx[:, None], _SHUF_DNUMS, slice_sizes=(1,),
                      mode=lax.GatherScatterMode.PROMISE_IN_BOUNDS)


@functools.partial(
    pl.kernel,
    mesh=_mesh,
    compiler_params=pltpu.CompilerParams(
        needs_layout_passes=False, skip_device_barrier=True),
    out_type=jax.ShapeDtypeStruct((B,), jnp.float32),
    scratch_types=[
        pltpu.VMEM((_BPW,), jnp.int32),
        pltpu.VMEM((_BPW,), jnp.int32),
        pltpu.VMEM((_NSLOT, 1, D), jnp.float32),
        pltpu.VMEM((_NSLOT, 1, D), jnp.float32),
        pltpu.VMEM((_BPW,), jnp.float32),
        [pltpu.SemaphoreType.DMA] * _NSLOT,
        [pltpu.SemaphoreType.DMA] * _NSLOT,
    ],
)
def _mf_kernel(u_hbm, i_hbm, U_hbm, V_hbm, out_hbm,
               uidx, vidx, ublk, vblk, outv, usem, vsem):
    wid = lax.axis_index("s") * _NC + lax.axis_index("c")
    base = wid * _BPW

    pltpu.sync_copy(u_hbm.at[pl.ds(base, _BPW)], uidx)
    pltpu.sync_copy(i_hbm.at[pl.ds(base, _BPW)], vidx)

    lanes = lax.iota(jnp.int32, _L)

    def _extract(vec, t):
        return jnp.sum(jnp.where(lanes == t, vec, 0))

    def issue(g, t, slot):
        ru = _extract(uidx[pl.ds(g * _L, _L)], t)
        rv = _extract(vidx[pl.ds(g * _L, _L)], t)
        pltpu.async_copy(U_hbm.at[ru >> 3, pl.ds(ru & 7, 1)],
                         ublk.at[slot], usem[slot])
        pltpu.async_copy(V_hbm.at[rv >> 3, pl.ds(rv & 7, 1)],
                         vblk.at[slot], vsem[slot])

    for t in range(_NSLOT):
        issue(0, t, t)

    def body_g(g, carry):
        acc = jnp.zeros((_L,), jnp.float32)
        for t in range(_L):
            slot = t % _NSLOT
            pltpu.make_async_copy(
                U_hbm.at[0, pl.ds(0, 1)], ublk.at[slot], usem[slot]).wait()
            pltpu.make_async_copy(
                V_hbm.at[0, pl.ds(0, 1)], vblk.at[slot], vsem[slot]).wait()
            p = ublk[slot, 0, pl.ds(0, _L)] * vblk[slot, 0, pl.ds(0, _L)]
            for c in range(1, D // _L):
                p += (ublk[slot, 0, pl.ds(c * _L, _L)]
                      * vblk[slot, 0, pl.ds(c * _L, _L)])
            for h in (8, 4, 2, 1):
                p = p + _lane_shuffle(p, lanes ^ h)
            acc = jnp.where(lanes == t, p, acc)

            if t < _NSLOT:
                issue(g, t + _NSLOT, slot)
            else:
                @pl.when(g < _NG - 1)
                def _():
                    issue(g + 1, t - _NSLOT, slot)

        outv[pl.ds(g * _L, _L)] = acc
        return carry

    lax.fori_loop(0, _NG, body_g, 0)
    pltpu.sync_copy(outv, out_hbm.at[pl.ds(base, _BPW)])


def kernel(u, i, U_emb, V_emb):
    U3 = U_emb.reshape(125000, 8, D)
    V3 = V_emb.reshape(125000, 8, D)
    return _mf_kernel(u.astype(jnp.int32), i.astype(jnp.int32), U3, V3)
